# baseline (speedup 1.0000x reference)
import functools

import jax
import jax.numpy as jnp
from jax import lax
from jax.experimental import pallas as pl
from jax.experimental.pallas import tpu as pltpu
from jax.experimental.pallas import tpu_sc as plsc

NUM_CLASSES = 1000000
FEAT_DIM = 64
BATCH = 16384

NC = 2
NS = 16
LANES = 16
NW = NC * NS
B_PER_W = BATCH // NW
N_CHUNK = 4
CHUNK = B_PER_W // N_CHUNK
VPR = FEAT_DIM // LANES


def _sc_body(feat_hbm, tgt_hbm, cent_hbm, out_hbm,
             idx_v, feat_v, rows_v, accv, gsem, fsem):
    wid = lax.axis_index("s") * NC + lax.axis_index("c")

    fcopy = pltpu.async_copy(feat_hbm.at[wid], feat_v, fsem)
    pltpu.sync_copy(tgt_hbm.at[wid], idx_v)

    gathers = [
        pltpu.async_copy(cent_hbm.at[idx_v.at[j]], rows_v.at[j], gsem)
        for j in range(N_CHUNK)
    ]
    fcopy.wait()
    for g in gathers:
        g.wait()

    accs = tuple(jnp.zeros((LANES,), jnp.float32) for _ in range(VPR))
    for j in range(N_CHUNK):
        def body(i, a, j=j):
            out = []
            for q in range(VPR):
                sl = pl.ds(q * LANES, LANES)
                d = feat_v[j, i, sl] - rows_v[j, i, sl]
                out.append(a[q] + d * d)
            return tuple(out)
        accs = lax.fori_loop(0, CHUNK, body, accs)

    accv[...] = (accs[0] + accs[1]) + (accs[2] + accs[3])
    pltpu.sync_copy(accv, out_hbm.at[wid])


_sc_partials = functools.partial(
    pl.kernel,
    out_type=jax.ShapeDtypeStruct((NW, LANES), jnp.float32),
    mesh=plsc.VectorSubcoreMesh(core_axis_name="c", subcore_axis_name="s"),
    compiler_params=pltpu.CompilerParams(use_tc_tiling_on_sc=False),
    scratch_types=[
        pltpu.VMEM((N_CHUNK, CHUNK), jnp.int32),
        pltpu.VMEM((N_CHUNK, CHUNK, FEAT_DIM), jnp.float32),
        pltpu.VMEM((N_CHUNK, CHUNK, FEAT_DIM), jnp.float32),
        pltpu.VMEM((LANES,), jnp.float32),
        pltpu.SemaphoreType.DMA,
        pltpu.SemaphoreType.DMA,
    ],
)(_sc_body)


def _finish_body(p_ref, o_ref):
    o_ref[0, 0] = jnp.sum(p_ref[...]) * (1.0 / BATCH)


_finish = pl.pallas_call(
    _finish_body,
    out_shape=jax.ShapeDtypeStruct((1, 1), jnp.float32),
    in_specs=[pl.BlockSpec(memory_space=pltpu.VMEM)],
    out_specs=pl.BlockSpec(memory_space=pltpu.SMEM),
)


def kernel(features, targets, centers):
    feat = features.reshape(NW, N_CHUNK, CHUNK, FEAT_DIM)
    tgt = targets.astype(jnp.int32).reshape(NW, N_CHUNK, CHUNK)
    partials = _sc_partials(feat, tgt, centers)
    return _finish(partials)[0, 0]
